# Initial kernel scaffold; baseline (speedup 1.0000x reference)
import jax
import jax.numpy as jnp
from jax.experimental import pallas as pl


def kernel(x, pos_embedding):
    raise NotImplementedError("write your pallas kernel here")


# speedup vs baseline: 2.3575x; 2.3575x over previous
import jax
import jax.numpy as jnp
from jax.experimental import pallas as pl


_SEQ_BLK = 512


def _body(x_ref, p_ref, o_ref):
    o_ref[...] = x_ref[...] + p_ref[...]


def kernel(x, pos_embedding):
    b, s, d = x.shape
    pos = jax.lax.slice(pos_embedding, (0, 0), (s, d))
    n_seq = s // _SEQ_BLK
    return pl.pallas_call(
        _body,
        grid=(n_seq, b),
        in_specs=[
            pl.BlockSpec((1, _SEQ_BLK, d), lambda j, i: (i, j, 0)),
            pl.BlockSpec((_SEQ_BLK, d), lambda j, i: (j, 0)),
        ],
        out_specs=pl.BlockSpec((1, _SEQ_BLK, d), lambda j, i: (i, j, 0)),
        out_shape=jax.ShapeDtypeStruct((b, s, d), x.dtype),
    )(x, pos)
